# baseline (speedup 1.0000x reference)
import jax
import jax.numpy as jnp
from jax.experimental import pallas as pl
from jax.experimental.pallas import tpu as pltpu

B, W, R, M = 256, 4, 8, 8192
BB = 16
MB = 4096


def _freeness_kernel(fg_ref, ww_ref, rw_ref, prev_ref, out_ref):
    for b in range(BB):
        prev = prev_ref[b]
        keep = 1.0 - ww_ref[b, 0]
        for w in range(1, W):
            keep = keep * (1.0 - ww_ref[b, w])
        usage = prev + (1.0 - prev) * (1.0 - keep)
        phi = 1.0 - fg_ref[b, 0] * rw_ref[b, 0]
        for r in range(1, R):
            phi = phi * (1.0 - fg_ref[b, r] * rw_ref[b, r])
        out_ref[b] = usage * phi


def kernel(write_weights, free_gate, read_weights, prev_usage):
    grid = (B // BB, M // MB)
    return pl.pallas_call(
        _freeness_kernel,
        grid=grid,
        in_specs=[
            pl.BlockSpec((BB, R), lambda i, j: (i, 0), memory_space=pltpu.SMEM),
            pl.BlockSpec((BB, W, MB), lambda i, j: (i, 0, j)),
            pl.BlockSpec((BB, R, MB), lambda i, j: (i, 0, j)),
            pl.BlockSpec((BB, MB), lambda i, j: (i, j)),
        ],
        out_specs=pl.BlockSpec((BB, MB), lambda i, j: (i, j)),
        out_shape=jax.ShapeDtypeStruct((B, M), jnp.float32),
    )(free_gate, write_weights, read_weights, prev_usage)


# speedup vs baseline: 1.1933x; 1.1933x over previous
import jax
import jax.numpy as jnp
from jax.experimental import pallas as pl
from jax.experimental.pallas import tpu as pltpu

B, W, R, M = 256, 4, 8, 8192
BB = 16
MH = M // 2


def _freeness_kernel(fg_ref, wwa_ref, wwb_ref, rwa_ref, rwb_ref,
                     prev_ref, out_ref):
    ww_halves = (wwa_ref, wwb_ref)
    rw_halves = (rwa_ref, rwb_ref)
    for b in range(BB):
        for h in range(2):
            sl = pl.ds(h * MH, MH)
            prev = prev_ref[b, sl]
            ww_h = ww_halves[h]
            rw_h = rw_halves[h]
            keep = 1.0 - ww_h[b, 0]
            for w in range(1, W):
                keep = keep * (1.0 - ww_h[b, w])
            usage = prev + (1.0 - prev) * (1.0 - keep)
            phi = 1.0 - fg_ref[b, 0] * rw_h[b, 0]
            for r in range(1, R):
                phi = phi * (1.0 - fg_ref[b, r] * rw_h[b, r])
            out_ref[b, sl] = usage * phi


def kernel(write_weights, free_gate, read_weights, prev_usage):
    grid = (B // BB,)
    return pl.pallas_call(
        _freeness_kernel,
        grid=grid,
        in_specs=[
            pl.BlockSpec((BB, R), lambda i: (i, 0), memory_space=pltpu.SMEM),
            pl.BlockSpec((BB, W, MH), lambda i: (i, 0, 0)),
            pl.BlockSpec((BB, W, MH), lambda i: (i, 0, 1)),
            pl.BlockSpec((BB, R, MH), lambda i: (i, 0, 0)),
            pl.BlockSpec((BB, R, MH), lambda i: (i, 0, 1)),
            pl.BlockSpec((BB, M), lambda i: (i, 0)),
        ],
        out_specs=pl.BlockSpec((BB, M), lambda i: (i, 0)),
        out_shape=jax.ShapeDtypeStruct((B, M), jnp.float32),
    )(free_gate, write_weights, write_weights, read_weights, read_weights,
      prev_usage)


# speedup vs baseline: 1.1943x; 1.0008x over previous
import jax
import jax.numpy as jnp
from jax.experimental import pallas as pl
from jax.experimental.pallas import tpu as pltpu

B, W, R, M = 256, 4, 8, 8192
BB = 16


def _freeness_kernel(fg_ref, ww_ref, rw_ref, prev_ref, out_ref):
    for b in range(BB):
        prev = prev_ref[b]
        keep = 1.0 - ww_ref[b, 0]
        for w in range(1, W):
            keep = keep * (1.0 - ww_ref[b, w])
        usage = prev + (1.0 - prev) * (1.0 - keep)
        phi = 1.0 - fg_ref[b, 0] * rw_ref[b, 0]
        for r in range(1, R):
            phi = phi * (1.0 - fg_ref[b, r] * rw_ref[b, r])
        out_ref[b] = usage * phi


def kernel(write_weights, free_gate, read_weights, prev_usage):
    grid = (B // BB,)
    return pl.pallas_call(
        _freeness_kernel,
        grid=grid,
        in_specs=[
            pl.BlockSpec((BB, R), lambda i: (i, 0), memory_space=pltpu.SMEM),
            pl.BlockSpec((BB, W, M), lambda i: (i, 0, 0)),
            pl.BlockSpec((BB, R, M), lambda i: (i, 0, 0)),
            pl.BlockSpec((BB, M), lambda i: (i, 0)),
        ],
        out_specs=pl.BlockSpec((BB, M), lambda i: (i, 0)),
        out_shape=jax.ShapeDtypeStruct((B, M), jnp.float32),
    )(free_gate, write_weights, read_weights, prev_usage)
